# baseline (speedup 1.0000x reference)
import functools

import jax
import jax.numpy as jnp
from jax import lax
from jax.experimental import pallas as pl
from jax.experimental.pallas import tpu as pltpu
from jax.experimental.pallas import tpu_sc as plsc

VOCAB = 1000000
D = 64
L_SEQ = 200
B = 1024
NC, NS = 2, 16
NW = NC * NS
BPW = B // NW
NBUF = 4
NOUT = BPW // NBUF


def _rope_rows(rows_v, cos_v, sin_v, b):
    @plsc.parallel_loop(0, L_SEQ, step=1, unroll=8)
    def row_body(r):
        rq, rr = r >> 3, r & 7
        c0 = cos_v[pl.ds(r * 32, 16)]
        c1 = cos_v[pl.ds(r * 32 + 16, 16)]
        s0 = sin_v[pl.ds(r * 32, 16)]
        s1 = sin_v[pl.ds(r * 32 + 16, 16)]
        x0 = rows_v[b, rq, rr, pl.ds(0, 16)]
        x1 = rows_v[b, rq, rr, pl.ds(16, 16)]
        x2 = rows_v[b, rq, rr, pl.ds(32, 16)]
        x3 = rows_v[b, rq, rr, pl.ds(48, 16)]
        rows_v[b, rq, rr, pl.ds(0, 16)] = x0 * c0 - x2 * s0
        rows_v[b, rq, rr, pl.ds(16, 16)] = x1 * c1 - x3 * s1
        rows_v[b, rq, rr, pl.ds(32, 16)] = x2 * c0 + x0 * s0
        rows_v[b, rq, rr, pl.ds(48, 16)] = x3 * c1 + x1 * s1


def _sc_body(idx_hbm, table_hbm, cos_hbm, sin_hbm, out_hbm,
             idx_v, rows_v, cos_v, sin_v,
             g0, g1, g2, g3, s0, s1, s2, s3):
    gsems = (g0, g1, g2, g3)
    ssems = (s0, s1, s2, s3)
    wid = lax.axis_index("s") * NC + lax.axis_index("c")
    bat0 = wid * BPW

    def issue_gather(j, b):
        def grp(g, _):
            vec = idx_v[j, pl.ds(g * 16, 16)]
            for k in range(16):
                v = vec[k]
                r = g * 16 + k
                pltpu.async_copy(table_hbm.at[v >> 3, v & 7],
                                 rows_v.at[b, r >> 3, r & 7], gsems[b])
            return _

        lax.fori_loop(0, L_SEQ // 16, grp, None)
        tail = L_SEQ - 16
        vec = idx_v[j, pl.ds(tail, 16)]
        for k in range(L_SEQ % 16, 16):
            v = vec[k]
            r = tail + k
            pltpu.async_copy(table_hbm.at[v >> 3, v & 7],
                             rows_v.at[b, r >> 3, r & 7], gsems[b])

    def issue_store(j, b):
        pltpu.async_copy(rows_v.at[b], out_hbm.at[bat0 + j], ssems[b])

    def wait_gather(b):
        pltpu.make_async_copy(out_hbm.at[bat0], rows_v.at[b], gsems[b]).wait()

    def wait_store(b):
        pltpu.make_async_copy(out_hbm.at[bat0], rows_v.at[b], ssems[b]).wait()

    pltpu.sync_copy(idx_hbm.at[pl.ds(bat0, BPW)], idx_v)
    pltpu.sync_copy(cos_hbm, cos_v)
    pltpu.sync_copy(sin_hbm, sin_v)

    issue_gather(0, 0)
    issue_gather(1, 1)

    for b in range(NBUF):
        wait_gather(b)
        _rope_rows(rows_v, cos_v, sin_v, b)
        issue_store(b, b)
        b2 = (b + 2) % NBUF
        if b >= 2:
            wait_store(b2)
        issue_gather(b + 2, b2)

    def outer_body(outer, _):
        for b in range(NBUF):
            j = outer * NBUF + b
            wait_gather(b)
            _rope_rows(rows_v, cos_v, sin_v, b)
            issue_store(j, b)
            b2 = (b + 2) % NBUF
            wait_store(b2)
            issue_gather(j + 2, b2)
        return _

    lax.fori_loop(1, NOUT - 1, outer_body, None)

    base = (NOUT - 1) * NBUF
    for b in range(NBUF):
        j = base + b
        wait_gather(b)
        _rope_rows(rows_v, cos_v, sin_v, b)
        issue_store(j, b)
        if b < 2:
            b2 = (b + 2) % NBUF
            wait_store(b2)
            issue_gather(j + 2, b2)

    for b in range(NBUF):
        wait_store(b)


def kernel(emb_inputs, src_emb_weight):
    table3 = src_emb_weight.reshape(VOCAB // 8, 8, D)
    inv_freq = 1.0 / (10000.0 ** (jnp.arange(0, D, 2, dtype=jnp.float32) / D))
    pos = jnp.arange(L_SEQ, dtype=jnp.float32)
    ang = pos[:, None] * inv_freq[None, :]
    cos_h = jnp.cos(ang).reshape(-1)
    sin_h = jnp.sin(ang).reshape(-1)

    mesh = plsc.VectorSubcoreMesh(core_axis_name="c", subcore_axis_name="s")
    fn = functools.partial(
        pl.kernel,
        out_type=jax.ShapeDtypeStruct((B, L_SEQ // 8, 8, D), jnp.float32),
        mesh=mesh,
        scratch_types=[
            pltpu.VMEM((BPW, L_SEQ), jnp.int32),
            pltpu.VMEM((NBUF, L_SEQ // 8, 8, D), jnp.float32),
            pltpu.VMEM((L_SEQ * D // 2,), jnp.float32),
            pltpu.VMEM((L_SEQ * D // 2,), jnp.float32),
        ] + [pltpu.SemaphoreType.DMA] * 8,
    )(_sc_body)
    out = fn(emb_inputs, table3, cos_h, sin_h)
    return out.reshape(B, L_SEQ, D)


# speedup vs baseline: 1.0077x; 1.0077x over previous
import functools

import jax
import jax.numpy as jnp
from jax import lax
from jax.experimental import pallas as pl
from jax.experimental.pallas import tpu as pltpu
from jax.experimental.pallas import tpu_sc as plsc

VOCAB = 1000000
D = 64
L_SEQ = 200
B = 1024
NC, NS = 2, 16
NW = NC * NS
BPW = B // NW
NBUF = 4
NOUT = BPW // NBUF


def _rope_rows(rows_v, cos_v, sin_v, b):
    @plsc.parallel_loop(0, L_SEQ, step=1, unroll=4)
    def row_body(r):
        rq, rr = r >> 3, r & 7
        c0 = cos_v[pl.ds(r * 32, 16)]
        c1 = cos_v[pl.ds(r * 32 + 16, 16)]
        s0 = sin_v[pl.ds(r * 32, 16)]
        s1 = sin_v[pl.ds(r * 32 + 16, 16)]
        x0 = rows_v[b, rq, rr, pl.ds(0, 16)]
        x1 = rows_v[b, rq, rr, pl.ds(16, 16)]
        x2 = rows_v[b, rq, rr, pl.ds(32, 16)]
        x3 = rows_v[b, rq, rr, pl.ds(48, 16)]
        rows_v[b, rq, rr, pl.ds(0, 16)] = x0 * c0 - x2 * s0
        rows_v[b, rq, rr, pl.ds(16, 16)] = x1 * c1 - x3 * s1
        rows_v[b, rq, rr, pl.ds(32, 16)] = x2 * c0 + x0 * s0
        rows_v[b, rq, rr, pl.ds(48, 16)] = x3 * c1 + x1 * s1


def _sc_body(idx_hbm, table_hbm, cos_hbm, sin_hbm, out_hbm,
             idx_v, rows_v, cos_v, sin_v,
             g0, g1, g2, g3, s0, s1, s2, s3):
    gsems = (g0, g1, g2, g3)
    ssems = (s0, s1, s2, s3)
    wid = lax.axis_index("s") * NC + lax.axis_index("c")
    bat0 = wid * BPW

    def issue_gather(j, b):
        def grp(g, _):
            vec = idx_v[j, pl.ds(g * 16, 16)]
            for k in range(16):
                v = vec[k]
                r = g * 16 + k
                pltpu.async_copy(table_hbm.at[v >> 3, v & 7],
                                 rows_v.at[b, r >> 3, r & 7], gsems[b])
            return _

        lax.fori_loop(0, L_SEQ // 16, grp, None)
        tail = L_SEQ - 16
        vec = idx_v[j, pl.ds(tail, 16)]
        for k in range(L_SEQ % 16, 16):
            v = vec[k]
            r = tail + k
            pltpu.async_copy(table_hbm.at[v >> 3, v & 7],
                             rows_v.at[b, r >> 3, r & 7], gsems[b])

    def issue_store(j, b):
        pltpu.async_copy(rows_v.at[b], out_hbm.at[bat0 + j], ssems[b])

    def wait_gather(b):
        pltpu.make_async_copy(out_hbm.at[bat0], rows_v.at[b], gsems[b]).wait()

    def wait_store(b):
        pltpu.make_async_copy(out_hbm.at[bat0], rows_v.at[b], ssems[b]).wait()

    pltpu.sync_copy(idx_hbm.at[pl.ds(bat0, BPW)], idx_v)
    pltpu.sync_copy(cos_hbm, cos_v)
    pltpu.sync_copy(sin_hbm, sin_v)

    issue_gather(0, 0)
    issue_gather(1, 1)

    for b in range(NBUF):
        wait_gather(b)
        b2 = (b + 2) % NBUF
        if b >= 2:
            wait_store(b2)
        issue_gather(b + 2, b2)
        _rope_rows(rows_v, cos_v, sin_v, b)
        issue_store(b, b)

    def outer_body(outer, _):
        for b in range(NBUF):
            j = outer * NBUF + b
            wait_gather(b)
            b2 = (b + 2) % NBUF
            wait_store(b2)
            issue_gather(j + 2, b2)
            _rope_rows(rows_v, cos_v, sin_v, b)
            issue_store(j, b)
        return _

    lax.fori_loop(1, NOUT - 1, outer_body, None)

    base = (NOUT - 1) * NBUF
    for b in range(NBUF):
        j = base + b
        wait_gather(b)
        if b < 2:
            b2 = (b + 2) % NBUF
            wait_store(b2)
            issue_gather(j + 2, b2)
        _rope_rows(rows_v, cos_v, sin_v, b)
        issue_store(j, b)

    for b in range(NBUF):
        wait_store(b)


def kernel(emb_inputs, src_emb_weight):
    table3 = src_emb_weight.reshape(VOCAB // 8, 8, D)
    inv_freq = 1.0 / (10000.0 ** (jnp.arange(0, D, 2, dtype=jnp.float32) / D))
    pos = jnp.arange(L_SEQ, dtype=jnp.float32)
    ang = pos[:, None] * inv_freq[None, :]
    cos_h = jnp.cos(ang).reshape(-1)
    sin_h = jnp.sin(ang).reshape(-1)

    mesh = plsc.VectorSubcoreMesh(core_axis_name="c", subcore_axis_name="s")
    fn = functools.partial(
        pl.kernel,
        out_type=jax.ShapeDtypeStruct((B, L_SEQ // 8, 8, D), jnp.float32),
        mesh=mesh,
        scratch_types=[
            pltpu.VMEM((BPW, L_SEQ), jnp.int32),
            pltpu.VMEM((NBUF, L_SEQ // 8, 8, D), jnp.float32),
            pltpu.VMEM((L_SEQ * D // 2,), jnp.float32),
            pltpu.VMEM((L_SEQ * D // 2,), jnp.float32),
        ] + [pltpu.SemaphoreType.DMA] * 8,
    )(_sc_body)
    out = fn(emb_inputs, table3, cos_h, sin_h)
    return out.reshape(B, L_SEQ, D)
